# baseline (speedup 1.0000x reference)
import functools

import jax
import jax.numpy as jnp
from jax.experimental import pallas as pl
from jax.experimental.pallas import tpu as pltpu


def _moe_body(x_ref, wgt_ref, we_ref, be_ref, y_ref, stats_ref, loss_ref,
              *, n_steps, n_csteps):
    e = wgt_ref.shape[0]
    x = x_ref[...]
    i = pl.program_id(0)
    j = pl.program_id(1)

    y_ref[...] = (jnp.dot(x, we_ref[...], preferred_element_type=jnp.float32)
                  + be_ref[...])

    @pl.when(j == 0)
    def _routing():
        logits_t = jax.lax.dot_general(
            wgt_ref[...], x, (((1,), (1,)), ((), ())),
            preferred_element_type=jnp.float32)

        row = jax.lax.broadcasted_iota(jnp.int32, logits_t.shape, 0)
        m1 = jnp.max(logits_t, axis=0, keepdims=True)
        idx1 = jnp.min(jnp.where(logits_t == m1, row, e), axis=0,
                       keepdims=True)
        masked = jnp.where(row == idx1, -jnp.inf, logits_t)
        m2 = jnp.max(masked, axis=0, keepdims=True)
        idx2 = jnp.min(jnp.where(masked == m2, row, e), axis=0, keepdims=True)

        q = jnp.exp(m2 - m1)
        s = 1.0 + q
        g1 = 1.0 / s
        g2 = q / s

        one1 = (row == idx1).astype(jnp.float32)
        one2 = (row == idx2).astype(jnp.float32)
        imp_part = jnp.sum(one1 * g1 + one2 * g2, axis=1, keepdims=True)
        load_part = jnp.sum(one1 + one2 * (g2 > 0).astype(jnp.float32),
                            axis=1, keepdims=True)

        @pl.when(i == 0)
        def _init():
            stats_ref[...] = jnp.zeros_like(stats_ref)

        stats_ref[:, 0:1] += imp_part
        stats_ref[:, 1:2] += load_part

    @pl.when((i == n_steps - 1) & (j == n_csteps - 1))
    def _finalize():
        def cv2(v):
            mean = jnp.sum(v) / e
            var = jnp.sum((v - mean) ** 2) / (e - 1)
            return var / (mean * mean + 1e-10)

        loss = cv2(stats_ref[:, 0:1]) + cv2(stats_ref[:, 1:2])
        loss_ref[...] = jnp.full((1, 1), loss, dtype=jnp.float32)


def kernel(x, w_gate, w_noise, W_e, b_e):
    del w_noise
    n, d = x.shape
    e = w_gate.shape[1]
    tn = 2048 if n % 2048 == 0 else n
    n_steps = n // tn
    n_csteps = 2 if d % 2 == 0 else 1
    td = d // n_csteps

    y, _, loss = pl.pallas_call(
        functools.partial(_moe_body, n_steps=n_steps, n_csteps=n_csteps),
        grid=(n_steps, n_csteps),
        in_specs=[
            pl.BlockSpec((tn, d), lambda i, j: (i, 0)),
            pl.BlockSpec((e, d), lambda i, j: (0, 0)),
            pl.BlockSpec((d, td), lambda i, j: (0, j)),
            pl.BlockSpec((1, td), lambda i, j: (0, j)),
        ],
        out_specs=[
            pl.BlockSpec((tn, td), lambda i, j: (i, j)),
            pl.BlockSpec((e, 2), lambda i, j: (0, 0)),
            pl.BlockSpec((1, 1), lambda i, j: (0, 0)),
        ],
        out_shape=[
            jax.ShapeDtypeStruct((n, d), jnp.float32),
            jax.ShapeDtypeStruct((e, 2), jnp.float32),
            jax.ShapeDtypeStruct((1, 1), jnp.float32),
        ],
        compiler_params=pltpu.CompilerParams(
            dimension_semantics=("arbitrary", "arbitrary")),
    )(x, w_gate.T, W_e, b_e.reshape(1, d))
    return y, loss[0, 0]


# speedup vs baseline: 1.5415x; 1.5415x over previous
import functools

import jax
import jax.numpy as jnp
from jax.experimental import pallas as pl
from jax.experimental.pallas import tpu as pltpu


def _moe_body(x_ref, wgt_ref, we_ref, be_ref, y_ref, stats_ref, loss_ref,
              *, n_steps):
    e = wgt_ref.shape[0]
    x = x_ref[...]

    y_ref[...] = (jnp.dot(x, we_ref[...], preferred_element_type=jnp.float32)
                  + be_ref[...])

    logits_t = jax.lax.dot_general(
        wgt_ref[...], x, (((1,), (1,)), ((), ())),
        preferred_element_type=jnp.float32)

    row = jax.lax.broadcasted_iota(jnp.int32, logits_t.shape, 0)
    m1 = jnp.max(logits_t, axis=0, keepdims=True)
    idx1 = jnp.min(jnp.where(logits_t == m1, row, e), axis=0, keepdims=True)
    masked = jnp.where(row == idx1, -jnp.inf, logits_t)
    m2 = jnp.max(masked, axis=0, keepdims=True)
    idx2 = jnp.min(jnp.where(masked == m2, row, e), axis=0, keepdims=True)

    q = jnp.exp(m2 - m1)
    s = 1.0 + q
    g1 = 1.0 / s
    g2 = q / s

    one1 = (row == idx1).astype(jnp.float32)
    one2 = (row == idx2).astype(jnp.float32)
    imp_part = jnp.sum(one1 * g1 + one2 * g2, axis=1, keepdims=True)
    load_part = jnp.sum(one1 + one2 * (g2 > 0).astype(jnp.float32),
                        axis=1, keepdims=True)

    @pl.when(pl.program_id(0) == 0)
    def _init():
        stats_ref[...] = jnp.zeros_like(stats_ref)

    stats_ref[:, 0:1] += imp_part
    stats_ref[:, 1:2] += load_part

    @pl.when(pl.program_id(0) == n_steps - 1)
    def _finalize():
        def cv2(v):
            mean = jnp.sum(v) / e
            var = jnp.sum((v - mean) ** 2) / (e - 1)
            return var / (mean * mean + 1e-10)

        loss = cv2(stats_ref[:, 0:1]) + cv2(stats_ref[:, 1:2])
        loss_ref[...] = jnp.full((1, 1), loss, dtype=jnp.float32)


def kernel(x, w_gate, w_noise, W_e, b_e):
    del w_noise
    n, d = x.shape
    e = w_gate.shape[1]
    tn = 2048 if n % 2048 == 0 else n
    n_steps = n // tn

    y, _, loss = pl.pallas_call(
        functools.partial(_moe_body, n_steps=n_steps),
        grid=(n_steps,),
        in_specs=[
            pl.BlockSpec((tn, d), lambda i: (i, 0)),
            pl.BlockSpec((e, d), lambda i: (0, 0)),
            pl.BlockSpec((d, d), lambda i: (0, 0)),
            pl.BlockSpec((1, d), lambda i: (0, 0)),
        ],
        out_specs=[
            pl.BlockSpec((tn, d), lambda i: (i, 0)),
            pl.BlockSpec((e, 2), lambda i: (0, 0)),
            pl.BlockSpec((1, 1), lambda i: (0, 0)),
        ],
        out_shape=[
            jax.ShapeDtypeStruct((n, d), jnp.float32),
            jax.ShapeDtypeStruct((e, 2), jnp.float32),
            jax.ShapeDtypeStruct((1, 1), jnp.float32),
        ],
        compiler_params=pltpu.CompilerParams(
            dimension_semantics=("arbitrary",)),
    )(x, w_gate.T, W_e, b_e.reshape(1, d))

    return y, loss[0, 0]
